# baseline (speedup 1.0000x reference)
import jax
import jax.numpy as jnp
from jax.experimental import pallas as pl
from jax.experimental.pallas import tpu as pltpu

B = 1024
D = 128
NW = 64
UB = 32


FC = 128


def _body(si_ref, sy_ref, ix_ref, states_ref, w_ref, b_ref,
          out_ref, acc, xbuf, wbuf, bgbuf, stage, sem, osem):
    g = pl.program_id(0)
    nblk = pl.num_programs(0)

    def issue(blk, slot):
        base = blk * UB
        for k in range(UB):
            si = si_ref[base + k]
            sy = sy_ref[base + k]
            pltpu.make_async_copy(states_ref.at[si], xbuf.at[slot, k],
                                  sem.at[slot]).start()
            pltpu.make_async_copy(w_ref.at[sy], wbuf.at[slot, k],
                                  sem.at[slot]).start()
            pltpu.make_async_copy(b_ref.at[sy], bgbuf.at[slot, k],
                                  sem.at[slot]).start()

    @pl.when(g == 0)
    def _init():
        acc[...] = jnp.zeros_like(acc)
        issue(0, 0)

    @pl.when(g + 1 < nblk)
    def _prefetch():
        issue(g + 1, (g + 1) % 2)

    slot = g % 2
    pltpu.make_async_copy(states_ref.at[pl.ds(0, UB)], xbuf.at[slot],
                          sem.at[slot]).wait()
    pltpu.make_async_copy(w_ref.at[pl.ds(0, UB)], wbuf.at[slot],
                          sem.at[slot]).wait()
    pltpu.make_async_copy(b_ref.at[pl.ds(0, UB)], bgbuf.at[slot],
                          sem.at[slot]).wait()

    for k in range(UB):
        w = wbuf[slot, k]
        xk = xbuf[slot, k]
        yt = jax.lax.dot_general(xk, w, (((0,), (1,)), ((), ())),
                                 preferred_element_type=jnp.float32)
        yt = yt + bgbuf[slot, k]
        sq = jnp.sum(yt * yt, axis=1, keepdims=True)
        yt = yt * jax.lax.rsqrt(jnp.maximum(sq, 1e-12))
        i = ix_ref[g * UB + k]
        acc[pl.ds(i, 1)] = acc[pl.ds(i, 1)] + yt[None]

    @pl.when(g == nblk - 1)
    def _flush():
        for c in range(B // FC):
            stage[...] = jnp.swapaxes(acc[c * FC:(c + 1) * FC], 1, 2)
            cp = pltpu.make_async_copy(
                stage, out_ref.at[pl.ds(c * FC, FC)], osem)
            cp.start()
            cp.wait()


def kernel(computed_states, W, b, indices, symbols, args):
    U = indices.shape[0]
    stacked_index = args[:, 0] * B + indices
    b2 = b[:, None, :]

    out = pl.pallas_call(
        _body,
        grid_spec=pltpu.PrefetchScalarGridSpec(
            num_scalar_prefetch=3,
            grid=(U // UB,),
            in_specs=[
                pl.BlockSpec(memory_space=pltpu.MemorySpace.HBM),
                pl.BlockSpec(memory_space=pltpu.MemorySpace.HBM),
                pl.BlockSpec(memory_space=pltpu.MemorySpace.HBM),
            ],
            out_specs=pl.BlockSpec(memory_space=pltpu.MemorySpace.HBM),
            scratch_shapes=[
                pltpu.VMEM((B, NW, D), jnp.float32),
                pltpu.VMEM((2, UB, D, NW), jnp.float32),
                pltpu.VMEM((2, UB, D, D), jnp.float32),
                pltpu.VMEM((2, UB, 1, D), jnp.float32),
                pltpu.VMEM((FC, D, NW), jnp.float32),
                pltpu.SemaphoreType.DMA((2,)),
                pltpu.SemaphoreType.DMA,
            ],
        ),
        out_shape=jax.ShapeDtypeStruct((B, D, NW), jnp.float32),
        compiler_params=pltpu.CompilerParams(
            dimension_semantics=("arbitrary",),
            vmem_limit_bytes=100 * 1024 * 1024,
        ),
    )(stacked_index, symbols, indices, computed_states, W, b2)
    return out


# speedup vs baseline: 1.0293x; 1.0293x over previous
import jax
import jax.numpy as jnp
from jax.experimental import pallas as pl
from jax.experimental.pallas import tpu as pltpu

B = 1024
D = 128
NW = 64
UB = 32


FC = 128


def _body(si_ref, sy_ref, ix_ref, states_ref, w_ref, b_ref,
          out_ref, acc, xbuf, wbuf, bgbuf, stage, sem, osem):
    g = pl.program_id(0)
    nblk = pl.num_programs(0)

    def issue(blk, slot):
        base = blk * UB
        for k in range(UB):
            si = si_ref[base + k]
            sy = sy_ref[base + k]
            pltpu.make_async_copy(states_ref.at[si], xbuf.at[slot, k],
                                  sem.at[slot]).start()
            pltpu.make_async_copy(w_ref.at[sy], wbuf.at[slot, k],
                                  sem.at[slot]).start()
            pltpu.make_async_copy(b_ref.at[sy], bgbuf.at[slot, k],
                                  sem.at[slot]).start()

    @pl.when(g == 0)
    def _init():
        acc[...] = jnp.zeros_like(acc)
        issue(0, 0)

    @pl.when(g + 1 < nblk)
    def _prefetch():
        issue(g + 1, (g + 1) % 2)

    slot = g % 2
    pltpu.make_async_copy(states_ref.at[pl.ds(0, UB)], xbuf.at[slot],
                          sem.at[slot]).wait()
    pltpu.make_async_copy(w_ref.at[pl.ds(0, UB)], wbuf.at[slot],
                          sem.at[slot]).wait()
    pltpu.make_async_copy(b_ref.at[pl.ds(0, UB)], bgbuf.at[slot],
                          sem.at[slot]).wait()

    for k in range(UB):
        w = wbuf[slot, k]
        xk = xbuf[slot, k].astype(jnp.bfloat16)
        yt = jax.lax.dot_general(xk, w, (((0,), (1,)), ((), ())),
                                 preferred_element_type=jnp.float32)
        yt = yt + bgbuf[slot, k]
        sq = jnp.sum(yt * yt, axis=1, keepdims=True)
        yt = yt * jax.lax.rsqrt(jnp.maximum(sq, 1e-12))
        i = ix_ref[g * UB + k]
        acc[pl.ds(i, 1)] = acc[pl.ds(i, 1)] + yt[None]

    @pl.when(g == nblk - 1)
    def _flush():
        for c in range(B // FC):
            stage[...] = jnp.swapaxes(acc[c * FC:(c + 1) * FC], 1, 2)
            cp = pltpu.make_async_copy(
                stage, out_ref.at[pl.ds(c * FC, FC)], osem)
            cp.start()
            cp.wait()


def kernel(computed_states, W, b, indices, symbols, args):
    U = indices.shape[0]
    stacked_index = args[:, 0] * B + indices
    w16 = W.astype(jnp.bfloat16)
    b2 = b[:, None, :]

    out = pl.pallas_call(
        _body,
        grid_spec=pltpu.PrefetchScalarGridSpec(
            num_scalar_prefetch=3,
            grid=(U // UB,),
            in_specs=[
                pl.BlockSpec(memory_space=pltpu.MemorySpace.HBM),
                pl.BlockSpec(memory_space=pltpu.MemorySpace.HBM),
                pl.BlockSpec(memory_space=pltpu.MemorySpace.HBM),
            ],
            out_specs=pl.BlockSpec(memory_space=pltpu.MemorySpace.HBM),
            scratch_shapes=[
                pltpu.VMEM((B, NW, D), jnp.float32),
                pltpu.VMEM((2, UB, D, NW), jnp.float32),
                pltpu.VMEM((2, UB, D, D), jnp.bfloat16),
                pltpu.VMEM((2, UB, 1, D), jnp.float32),
                pltpu.VMEM((FC, D, NW), jnp.float32),
                pltpu.SemaphoreType.DMA((2,)),
                pltpu.SemaphoreType.DMA,
            ],
        ),
        out_shape=jax.ShapeDtypeStruct((B, D, NW), jnp.float32),
        compiler_params=pltpu.CompilerParams(
            dimension_semantics=("arbitrary",),
            vmem_limit_bytes=100 * 1024 * 1024,
        ),
    )(stacked_index, symbols, indices, computed_states, w16, b2)
    return out


# speedup vs baseline: 1.4068x; 1.3668x over previous
import jax
import jax.numpy as jnp
from jax.experimental import pallas as pl
from jax.experimental.pallas import tpu as pltpu

B = 1024
D = 128
NW = 64
UB = 32


NSLOT = 3


def _body(si_ref, sy_ref, ix_ref, states_ref, w_ref, b_ref,
          out_ref, acc, xbuf, wbuf, bgbuf, sem, osem):
    g = pl.program_id(0)
    nblk = pl.num_programs(0)

    def issue(blk, slot):
        base = blk * UB
        for k in range(UB):
            si = si_ref[base + k]
            sy = sy_ref[base + k]
            pltpu.make_async_copy(states_ref.at[si], xbuf.at[slot, k],
                                  sem.at[slot]).start()
            pltpu.make_async_copy(w_ref.at[sy], wbuf.at[slot, k],
                                  sem.at[slot]).start()
            pltpu.make_async_copy(b_ref.at[sy], bgbuf.at[slot, k],
                                  sem.at[slot]).start()

    @pl.when(g == 0)
    def _init():
        acc[...] = jnp.zeros_like(acc)
        issue(0, 0)
        issue(1, 1)

    @pl.when(g + 2 < nblk)
    def _prefetch():
        issue(g + 2, (g + 2) % NSLOT)

    slot = g % NSLOT
    pltpu.make_async_copy(states_ref.at[pl.ds(0, UB)], xbuf.at[slot],
                          sem.at[slot]).wait()
    pltpu.make_async_copy(w_ref.at[pl.ds(0, UB)], wbuf.at[slot],
                          sem.at[slot]).wait()
    pltpu.make_async_copy(b_ref.at[pl.ds(0, UB)], bgbuf.at[slot],
                          sem.at[slot]).wait()

    for k in range(UB):
        w = wbuf[slot, k]
        xk = xbuf[slot, k].astype(jnp.bfloat16)
        yt = jax.lax.dot_general(xk, w, (((0,), (1,)), ((), ())),
                                 preferred_element_type=jnp.float32)
        yt = yt + bgbuf[slot, k]
        sq = jnp.sum(yt * yt, axis=1, keepdims=True)
        yt = yt * jax.lax.rsqrt(jnp.maximum(sq, 1e-12))
        i = ix_ref[g * UB + k]
        acc[pl.ds(i, 1)] = acc[pl.ds(i, 1)] + yt[None]

    @pl.when(g == nblk - 1)
    def _flush():
        pltpu.make_async_copy(acc, out_ref, osem).start()
        pltpu.make_async_copy(acc, out_ref, osem).wait()


def kernel(computed_states, W, b, indices, symbols, args):
    U = indices.shape[0]
    stacked_index = args[:, 0] * B + indices
    w16 = W.astype(jnp.bfloat16)
    b2 = b[:, None, :]

    out = pl.pallas_call(
        _body,
        grid_spec=pltpu.PrefetchScalarGridSpec(
            num_scalar_prefetch=3,
            grid=(U // UB,),
            in_specs=[
                pl.BlockSpec(memory_space=pltpu.MemorySpace.HBM),
                pl.BlockSpec(memory_space=pltpu.MemorySpace.HBM),
                pl.BlockSpec(memory_space=pltpu.MemorySpace.HBM),
            ],
            out_specs=pl.BlockSpec(memory_space=pltpu.MemorySpace.HBM),
            scratch_shapes=[
                pltpu.VMEM((B, NW, D), jnp.float32),
                pltpu.VMEM((NSLOT, UB, D, NW), jnp.float32),
                pltpu.VMEM((NSLOT, UB, D, D), jnp.bfloat16),
                pltpu.VMEM((NSLOT, UB, 1, D), jnp.float32),
                pltpu.SemaphoreType.DMA((NSLOT,)),
                pltpu.SemaphoreType.DMA,
            ],
        ),
        out_shape=jax.ShapeDtypeStruct((B, NW, D), jnp.float32),
        compiler_params=pltpu.CompilerParams(
            dimension_semantics=("arbitrary",),
            vmem_limit_bytes=100 * 1024 * 1024,
        ),
    )(stacked_index, symbols, indices, computed_states, w16, b2)
    return jnp.swapaxes(out, 1, 2)


# speedup vs baseline: 1.4638x; 1.0405x over previous
import jax
import jax.numpy as jnp
from jax.experimental import pallas as pl
from jax.experimental.pallas import tpu as pltpu

B = 1024
D = 128
NW = 64
UB = 64


NSLOT = 3


def _body(si_ref, sy_ref, ix_ref, states_ref, w_ref, b_ref,
          out_ref, acc, xbuf, wbuf, bgbuf, sem, osem):
    g = pl.program_id(0)
    nblk = pl.num_programs(0)

    def issue(blk, slot):
        base = blk * UB
        for k in range(UB):
            si = si_ref[base + k]
            sy = sy_ref[base + k]
            pltpu.make_async_copy(states_ref.at[si], xbuf.at[slot, k],
                                  sem.at[slot]).start()
            pltpu.make_async_copy(w_ref.at[sy], wbuf.at[slot, k],
                                  sem.at[slot]).start()
            pltpu.make_async_copy(b_ref.at[sy], bgbuf.at[slot, k],
                                  sem.at[slot]).start()

    @pl.when(g == 0)
    def _init():
        acc[...] = jnp.zeros_like(acc)
        issue(0, 0)
        issue(1, 1)

    @pl.when(g + 2 < nblk)
    def _prefetch():
        issue(g + 2, (g + 2) % NSLOT)

    slot = g % NSLOT
    pltpu.make_async_copy(states_ref.at[pl.ds(0, UB)], xbuf.at[slot],
                          sem.at[slot]).wait()
    pltpu.make_async_copy(w_ref.at[pl.ds(0, UB)], wbuf.at[slot],
                          sem.at[slot]).wait()
    pltpu.make_async_copy(b_ref.at[pl.ds(0, UB)], bgbuf.at[slot],
                          sem.at[slot]).wait()

    for k in range(UB):
        w = wbuf[slot, k]
        xk = xbuf[slot, k].astype(jnp.bfloat16)
        yt = jax.lax.dot_general(xk, w, (((0,), (1,)), ((), ())),
                                 preferred_element_type=jnp.float32)
        yt = yt + bgbuf[slot, k]
        sq = jnp.sum(yt * yt, axis=1, keepdims=True)
        yt = yt * jax.lax.rsqrt(jnp.maximum(sq, 1e-12))
        i = ix_ref[g * UB + k]
        acc[pl.ds(i, 1)] = acc[pl.ds(i, 1)] + yt[None]

    @pl.when(g == nblk - 1)
    def _flush():
        pltpu.make_async_copy(acc, out_ref, osem).start()
        pltpu.make_async_copy(acc, out_ref, osem).wait()


def kernel(computed_states, W, b, indices, symbols, args):
    U = indices.shape[0]
    stacked_index = args[:, 0] * B + indices
    w16 = W.astype(jnp.bfloat16)
    b2 = b[:, None, :]

    out = pl.pallas_call(
        _body,
        grid_spec=pltpu.PrefetchScalarGridSpec(
            num_scalar_prefetch=3,
            grid=(U // UB,),
            in_specs=[
                pl.BlockSpec(memory_space=pltpu.MemorySpace.HBM),
                pl.BlockSpec(memory_space=pltpu.MemorySpace.HBM),
                pl.BlockSpec(memory_space=pltpu.MemorySpace.HBM),
            ],
            out_specs=pl.BlockSpec(memory_space=pltpu.MemorySpace.HBM),
            scratch_shapes=[
                pltpu.VMEM((B, NW, D), jnp.float32),
                pltpu.VMEM((NSLOT, UB, D, NW), jnp.float32),
                pltpu.VMEM((NSLOT, UB, D, D), jnp.bfloat16),
                pltpu.VMEM((NSLOT, UB, 1, D), jnp.float32),
                pltpu.SemaphoreType.DMA((NSLOT,)),
                pltpu.SemaphoreType.DMA,
            ],
        ),
        out_shape=jax.ShapeDtypeStruct((B, NW, D), jnp.float32),
        compiler_params=pltpu.CompilerParams(
            dimension_semantics=("arbitrary",),
            vmem_limit_bytes=100 * 1024 * 1024,
        ),
    )(stacked_index, symbols, indices, computed_states, w16, b2)
    return jnp.swapaxes(out, 1, 2)
